# Initial kernel scaffold; baseline (speedup 1.0000x reference)
import jax
import jax.numpy as jnp
from jax.experimental import pallas as pl


def kernel(input, table):
    raise NotImplementedError("write your pallas kernel here")


# speedup vs baseline: 2.8268x; 2.8268x over previous
import functools
import math

import jax
import jax.numpy as jnp
from jax import lax
from jax.experimental import pallas as pl
from jax.experimental.pallas import tpu as pltpu
from jax.experimental.pallas import tpu_sc as plsc

EMBED_DIM = 128
SCALE = float(math.sqrt(EMBED_DIM))

NC = 2
NS = 16
NW = NC * NS
LANES = 16

B_TOTAL = 4096 * 50
B_PER_W = B_TOTAL // NW
CHUNK = 128
NCHUNK = B_PER_W // CHUNK
NBUF = 5
N_OUTER = NCHUNK // NBUF


def _emb_body(idx_hbm, table_hbm, out_hbm, idx_v, rows, gsem, ssem):
    c = lax.axis_index("c")
    s = lax.axis_index("s")
    wid = s * NC + c
    base = wid * B_PER_W

    pltpu.sync_copy(idx_hbm.at[wid], idx_v)

    def gather_start(g, buf):
        pltpu.async_copy(table_hbm.at[idx_v.at[g]], rows[buf], gsem)

    def gather_wait():
        pltpu.make_async_copy(table_hbm.at[idx_v.at[0]], rows[0], gsem).wait()

    def scatter_start(g, buf):
        pltpu.async_copy(rows[buf], out_hbm.at[pl.ds(base + g * CHUNK, CHUNK)],
                         ssem)

    def scatter_wait():
        pltpu.make_async_copy(rows[0], out_hbm.at[pl.ds(base, CHUNK)],
                              ssem).wait()

    def scale(buf):
        @pl.loop(0, CHUNK)
        def _(r):
            for col in range(EMBED_DIM // LANES):
                sl = pl.ds(col * LANES, LANES)
                rows[buf][r, sl] = rows[buf][r, sl] * SCALE

    gather_start(0, 0)

    @pl.loop(0, N_OUTER)
    def _(o):
        for b in range(NBUF):
            g = o * NBUF + b
            gather_wait()
            if b == NBUF - 1:
                scatter_wait()
            else:
                @pl.when(o > 0)
                def _():
                    scatter_wait()
            if b == NBUF - 1:
                @pl.when(o < N_OUTER - 1)
                def _():
                    gather_start(g + 1, 0)
            else:
                gather_start(g + 1, b + 1)
            scale(b)
            scatter_start(g, b)

    for _ in range(NBUF - 1):
        scatter_wait()


@jax.jit
def _emb_call(idx, table):
    mesh = plsc.VectorSubcoreMesh(core_axis_name="c", subcore_axis_name="s",
                                  num_cores=NC, num_subcores=NS)
    fn = pl.kernel(
        _emb_body,
        out_type=jax.ShapeDtypeStruct((B_TOTAL, EMBED_DIM), jnp.float32),
        mesh=mesh,
        scratch_types=[
            pltpu.VMEM((NCHUNK, CHUNK), jnp.int32),
            [pltpu.VMEM((CHUNK, EMBED_DIM), jnp.float32) for _ in range(NBUF)],
            pltpu.SemaphoreType.DMA,
            pltpu.SemaphoreType.DMA,
        ],
    )
    return fn(idx, table)


def kernel(input, table):
    idx = jnp.asarray(input, jnp.int32).reshape(NW, NCHUNK, CHUNK)
    out = _emb_call(idx, table)
    return out.reshape(input.shape[0], input.shape[1], EMBED_DIM)


# speedup vs baseline: 2.9720x; 1.0514x over previous
import functools
import math

import jax
import jax.numpy as jnp
from jax import lax
from jax.experimental import pallas as pl
from jax.experimental.pallas import tpu as pltpu
from jax.experimental.pallas import tpu_sc as plsc

EMBED_DIM = 128
SCALE = float(math.sqrt(EMBED_DIM))

NC = 2
NS = 16
NW = NC * NS
LANES = 16

B_TOTAL = 4096 * 50
B_PER_W = B_TOTAL // NW
CHUNK = 128
NCHUNK = B_PER_W // CHUNK
NBUF = 5
N_OUTER = NCHUNK // NBUF
GAHEAD = 3


def _emb_body(idx_hbm, table_hbm, out_hbm, idx_v, rows, gsem, ssem):
    c = lax.axis_index("c")
    s = lax.axis_index("s")
    wid = s * NC + c
    base = wid * B_PER_W

    pltpu.sync_copy(idx_hbm.at[wid], idx_v)

    def gather_start(g, buf):
        pltpu.async_copy(table_hbm.at[idx_v.at[g]], rows[buf], gsem)

    def gather_wait():
        pltpu.make_async_copy(table_hbm.at[idx_v.at[0]], rows[0], gsem).wait()

    def scatter_start(g, buf):
        pltpu.async_copy(rows[buf], out_hbm.at[pl.ds(base + g * CHUNK, CHUNK)],
                         ssem)

    def scatter_wait():
        pltpu.make_async_copy(rows[0], out_hbm.at[pl.ds(base, CHUNK)],
                              ssem).wait()

    def scale(buf):
        @pl.loop(0, CHUNK, unroll=8)
        def _(r):
            for col in range(EMBED_DIM // LANES):
                sl = pl.ds(col * LANES, LANES)
                rows[buf][r, sl] = rows[buf][r, sl] * SCALE

    for g in range(GAHEAD):
        gather_start(g, g)

    @pl.loop(0, N_OUTER)
    def _(o):
        for b in range(NBUF):
            g = o * NBUF + b
            gather_wait()
            if b >= NBUF - GAHEAD:
                scatter_wait()
            else:
                @pl.when(o > 0)
                def _():
                    scatter_wait()
            if NBUF * (N_OUTER - 1) + b + GAHEAD < NCHUNK:
                gather_start(g + GAHEAD, (b + GAHEAD) % NBUF)
            else:
                @pl.when(o < N_OUTER - 1)
                def _():
                    gather_start(g + GAHEAD, (b + GAHEAD) % NBUF)
            scale(b)
            scatter_start(g, b)

    for _ in range(NBUF - GAHEAD):
        scatter_wait()


@jax.jit
def _emb_call(idx, table):
    mesh = plsc.VectorSubcoreMesh(core_axis_name="c", subcore_axis_name="s",
                                  num_cores=NC, num_subcores=NS)
    fn = pl.kernel(
        _emb_body,
        out_type=jax.ShapeDtypeStruct((B_TOTAL, EMBED_DIM), jnp.float32),
        mesh=mesh,
        scratch_types=[
            pltpu.VMEM((NCHUNK, CHUNK), jnp.int32),
            [pltpu.VMEM((CHUNK, EMBED_DIM), jnp.float32) for _ in range(NBUF)],
            pltpu.SemaphoreType.DMA,
            pltpu.SemaphoreType.DMA,
        ],
    )
    return fn(idx, table)


def kernel(input, table):
    idx = jnp.asarray(input, jnp.int32).reshape(NW, NCHUNK, CHUNK)
    out = _emb_call(idx, table)
    return out.reshape(input.shape[0], input.shape[1], EMBED_DIM)
